# baseline (speedup 1.0000x reference)
import functools

import jax
import jax.numpy as jnp
from jax import lax
from jax.experimental import pallas as pl
from jax.experimental.pallas import tpu as pltpu
from jax.experimental.pallas import tpu_sc as plsc

N, E, D, H, OUT, G = 10000, 320000, 128, 128, 128, 64
NC, NS = 2, 16
NW = NC * NS
RB = 312
ACC_ROWS = 336
TRASH = 328
SCH = 2000
NSCH = E // SCH
GCH = 128
QCAP = SCH + GCH + 48

@functools.lru_cache(maxsize=1)
def _get_mp():
    mesh = plsc.VectorSubcoreMesh(core_axis_name="c", subcore_axis_name="s",
                                  num_cores=NC, num_subcores=NS)

    @functools.partial(
        pl.kernel,
        out_type=jax.ShapeDtypeStruct((N, H), jnp.float32),
        mesh=mesh,
        scratch_types=[
            pltpu.VMEM((2 * SCH,), jnp.int32),
            pltpu.VMEM((2 * SCH,), jnp.int32),
            pltpu.VMEM((QCAP,), jnp.int32),
            pltpu.VMEM((QCAP,), jnp.int32),
            pltpu.VMEM((GCH, H), jnp.float32),
            pltpu.VMEM((ACC_ROWS, H), jnp.float32),
            pltpu.SemaphoreType.DMA,
        ],
    )
    def _mp(h_hbm, src_hbm, dst_hbm, out_hbm, dstb_v, srcb_v, qsrc_v, qdst_v,
            rows_v, acc_v, sem):
        c = lax.axis_index("c")
        s = lax.axis_index("s")
        t = c * NS + s
        lo = t * RB
        hi = lo + RB + jnp.where(t == NW - 1, N - NW * RB, 0)
        z16 = jnp.zeros((16,), jnp.float32)

        def zero_body(i, carry):
            for v in range(8):
                acc_v[i, pl.ds(v * 16, 16)] = z16
            return carry

        lax.fori_loop(0, ACC_ROWS, zero_body, 0)

        def flush_group(g, carry):
            pltpu.async_copy(h_hbm.at[qsrc_v.at[pl.ds(g * GCH, GCH)]],
                             rows_v, sem).wait()

            def add_body(i, carry2):
                r = qdst_v[pl.ds(g * GCH + i, 16)][0]
                for v in range(8):
                    sl = pl.ds(v * 16, 16)
                    acc_v[r, sl] = acc_v[r, sl] + rows_v[i, sl]
                return carry2

            lax.fori_loop(0, GCH, add_body, 0)
            return carry

        def chunk_body(ci, qn):
            b = lax.rem(ci, 2)
            off = ci * SCH
            pltpu.sync_copy(dst_hbm.at[pl.ds(off, SCH)], dstb_v.at[pl.ds(b * SCH, SCH)])
            pltpu.sync_copy(src_hbm.at[pl.ds(off, SCH)], srcb_v.at[pl.ds(b * SCH, SCH)])

            def scan_body(k, qn2):
                dv = dstb_v[pl.ds(b * SCH + k * 16, 16)]
                sv = srcb_v[pl.ds(b * SCH + k * 16, 16)]
                for v in range(16):
                    dvv = dv[v]
                    svv = sv[v]
                    mv = (dvv >= lo) & (dvv < hi)

                    @pl.when(mv)
                    def _(qn2=qn2, svv=svv, dvv=dvv):
                        qsrc_v[pl.ds(qn2, 16)] = jnp.full((16,), svv, jnp.int32)
                        qdst_v[pl.ds(qn2, 16)] = jnp.full((16,), dvv - lo,
                                                          jnp.int32)

                    qn2 = qn2 + jnp.where(mv, 1, 0)
                return qn2

            qn = lax.fori_loop(0, SCH // 16, scan_body, qn)
            groups = qn // GCH
            lax.fori_loop(0, groups, flush_group, 0)
            tb = groups * GCH
            for v in range(8):
                sl = pl.ds(v * 16, 16)
                qsrc_v[sl] = qsrc_v[pl.ds(tb + v * 16, 16)]
                qdst_v[sl] = qdst_v[pl.ds(tb + v * 16, 16)]
            return qn - tb

        qn = lax.fori_loop(0, NSCH, chunk_body, 0)

        ztr = jnp.full((16,), TRASH, jnp.int32)
        zsrc = jnp.zeros((16,), jnp.int32)
        for v in range(8):
            qsrc_v[pl.ds(qn + v * 16, 16)] = zsrc
            qdst_v[pl.ds(qn + v * 16, 16)] = ztr
        lax.fori_loop(0, (qn + GCH - 1) // GCH, flush_group, 0)

        pltpu.sync_copy(acc_v.at[pl.ds(0, RB)], out_hbm.at[pl.ds(t * RB, RB)])

        @pl.when(t == NW - 1)
        def _():
            pltpu.sync_copy(acc_v.at[pl.ds(RB, N - NW * RB)],
                            out_hbm.at[pl.ds(NW * RB, N - NW * RB)])

    return _mp


BLK = 1000


def _layer_body(h_ref, p_ref, b_ref, W1_ref, b1_ref, W2_ref, b2_ref, Wj_ref,
                hn_ref, P_ref):
    i = pl.program_id(0)
    t = h_ref[...] + p_ref[...]
    a = jnp.maximum(jnp.dot(t, W1_ref[...],
                            preferred_element_type=jnp.float32) + b1_ref[...], 0.0)
    hn = jnp.maximum(jnp.dot(a, W2_ref[...],
                             preferred_element_type=jnp.float32) + b2_ref[...], 0.0)
    hn_ref[...] = hn
    hj = jnp.dot(hn, Wj_ref[...], preferred_element_type=jnp.float32)
    mask = (b_ref[0] == lax.broadcasted_iota(jnp.int32, (G, BLK), 0)
            ).astype(jnp.float32)
    Pc = jnp.dot(mask, hj, preferred_element_type=jnp.float32,
                 precision=lax.Precision.HIGHEST)

    @pl.when(i == 0)
    def _():
        P_ref[...] = Pc

    @pl.when(i > 0)
    def _():
        P_ref[...] = P_ref[...] + Pc


_layer = pl.pallas_call(
    _layer_body,
    grid=(N // BLK,),
    in_specs=[
        pl.BlockSpec((BLK, H), lambda i: (i, 0)),
        pl.BlockSpec((BLK, H), lambda i: (i, 0)),
        pl.BlockSpec((1, 1, BLK), lambda i: (i, 0, 0)),
        pl.BlockSpec((H, H), lambda i: (0, 0)),
        pl.BlockSpec((1, H), lambda i: (0, 0)),
        pl.BlockSpec((H, H), lambda i: (0, 0)),
        pl.BlockSpec((1, H), lambda i: (0, 0)),
        pl.BlockSpec((H, H), lambda i: (0, 0)),
    ],
    out_specs=[
        pl.BlockSpec((BLK, H), lambda i: (i, 0)),
        pl.BlockSpec((G, H), lambda i: (0, 0)),
    ],
    out_shape=[
        jax.ShapeDtypeStruct((N, H), jnp.float32),
        jax.ShapeDtypeStruct((G, H), jnp.float32),
    ],
)


def _head_body(P1_ref, P2_ref, P3_ref, b_ref, bjk_ref, Wc1_ref, bc1_ref,
               g_ref, bt_ref, Wc2_ref, bc2_ref, out_ref):
    mask = (b_ref[...] == lax.broadcasted_iota(jnp.int32, (G, N), 0)
            ).astype(jnp.float32)
    counts = jnp.sum(mask, axis=1, keepdims=True)
    pooled = P1_ref[...] + P2_ref[...] + P3_ref[...] + counts * bjk_ref[...]
    z = jnp.dot(pooled, Wc1_ref[...], preferred_element_type=jnp.float32) + bc1_ref[...]
    mean = jnp.mean(z, axis=0, keepdims=True)
    var = jnp.mean((z - mean) ** 2, axis=0, keepdims=True)
    z = (z - mean) / jnp.sqrt(var + 1e-5) * g_ref[...] + bt_ref[...]
    z = jnp.maximum(z, 0.0)
    out_ref[...] = jnp.dot(z, Wc2_ref[...], preferred_element_type=jnp.float32) + bc2_ref[...]


_head = pl.pallas_call(
    _head_body,
    out_shape=jax.ShapeDtypeStruct((G, OUT), jnp.float32),
)


def kernel(x, edge_index, batch,
           W1_0, b1_0, W2_0, b2_0,
           W1_1, b1_1, W2_1, b2_1,
           W1_2, b1_2, W2_2, b2_2,
           Wjk, bjk, Wc1, bc1, bn_gamma, bn_beta, Wc2, bc2):
    src = edge_index[0]
    dst = edge_index[1]
    b2d = batch.reshape(1, N)
    b3d = batch.reshape(N // BLK, 1, BLK)
    mp = _get_mp()
    params = [(W1_0, b1_0, W2_0, b2_0),
              (W1_1, b1_1, W2_1, b2_1),
              (W1_2, b1_2, W2_2, b2_2)]
    h = x
    Ps = []
    for l, (W1, b1, W2, b2) in enumerate(params):
        agg = mp(h, src, dst)
        h, P = _layer(h, agg, b3d, W1, b1.reshape(1, H), W2, b2.reshape(1, H),
                      Wjk[l * H:(l + 1) * H])
        Ps.append(P)
    return _head(Ps[0], Ps[1], Ps[2], b2d, bjk.reshape(1, H),
                 Wc1, bc1.reshape(1, H), bn_gamma.reshape(1, H),
                 bn_beta.reshape(1, H), Wc2, bc2.reshape(1, OUT))


# speedup vs baseline: 1.1504x; 1.1504x over previous
import functools

import jax
import jax.numpy as jnp
from jax import lax
from jax.experimental import pallas as pl
from jax.experimental.pallas import tpu as pltpu
from jax.experimental.pallas import tpu_sc as plsc

N, E, D, H, OUT, G = 10000, 320000, 128, 128, 128, 64
NC, NS = 2, 16
NW = NC * NS
RB = 312
ACC_ROWS = 336
TRASH = 328
SCH = 2000
NSCH = E // SCH
GCH = 128
QCAP = SCH + GCH + 64
DUMP = QCAP - 16

@functools.lru_cache(maxsize=1)
def _get_mp():
    mesh = plsc.VectorSubcoreMesh(core_axis_name="c", subcore_axis_name="s",
                                  num_cores=NC, num_subcores=NS)

    @functools.partial(
        pl.kernel,
        out_type=jax.ShapeDtypeStruct((N, H), jnp.float32),
        mesh=mesh,
        scratch_types=[
            pltpu.VMEM((2 * SCH,), jnp.int32),
            pltpu.VMEM((2 * SCH,), jnp.int32),
            pltpu.VMEM((QCAP,), jnp.int32),
            pltpu.VMEM((QCAP,), jnp.int32),
            pltpu.VMEM((GCH, H), jnp.float32),
            pltpu.VMEM((ACC_ROWS, H), jnp.float32),
            pltpu.SMEM((1,), jnp.int32),
            pltpu.SemaphoreType.DMA,
        ],
    )
    def _mp(h_hbm, src_hbm, dst_hbm, out_hbm, dstb_v, srcb_v, qsrc_v, qdst_v,
            rows_v, acc_v, qn_s, sem):
        c = lax.axis_index("c")
        s = lax.axis_index("s")
        t = c * NS + s
        lo = t * RB
        hi = lo + RB + jnp.where(t == NW - 1, N - NW * RB, 0)
        z16 = jnp.zeros((16,), jnp.float32)

        def zero_body(i, carry):
            for v in range(8):
                acc_v[i, pl.ds(v * 16, 16)] = z16
            return carry

        lax.fori_loop(0, ACC_ROWS, zero_body, 0)
        qn_s[0] = 0

        def flush_group(g, carry):
            pltpu.async_copy(h_hbm.at[qsrc_v.at[pl.ds(g * GCH, GCH)]],
                             rows_v, sem).wait()

            def add_body(i, carry2):
                r = qdst_v[pl.ds(g * GCH + i, 16)][0]
                for v in range(8):
                    sl = pl.ds(v * 16, 16)
                    acc_v[r, sl] = acc_v[r, sl] + rows_v[i, sl]
                return carry2

            lax.fori_loop(0, GCH, add_body, 0)
            return carry

        def chunk_body(ci, carry_c):
            b = lax.rem(ci, 2)
            off = ci * SCH
            pltpu.sync_copy(dst_hbm.at[pl.ds(off, SCH)], dstb_v.at[pl.ds(b * SCH, SCH)])
            pltpu.sync_copy(src_hbm.at[pl.ds(off, SCH)], srcb_v.at[pl.ds(b * SCH, SCH)])

            spanu = (hi - lo).astype(jnp.uint32)

            def scan_body(k, carry):
                base = b * SCH + k * 16
                dv = dstb_v[pl.ds(base, 16)]
                sv = srcb_v[pl.ds(base, 16)]
                mi = jnp.where((dv - lo).astype(jnp.uint32) < spanu, 1, 0)
                q = qn_s[0]
                for v in range(16):
                    miv = mi[v]
                    qidx = jnp.where(miv > 0, q, DUMP)
                    qsrc_v[pl.ds(qidx, 16)] = jnp.full((16,), sv[v], jnp.int32)
                    qdst_v[pl.ds(qidx, 16)] = jnp.full((16,), dv[v] - lo,
                                                       jnp.int32)
                    q = q + miv
                qn_s[0] = q
                return carry

            lax.fori_loop(0, SCH // 16, scan_body, 0)
            qn = qn_s[0]
            groups = qn // GCH
            lax.fori_loop(0, groups, flush_group, 0)
            tb = groups * GCH
            for v in range(8):
                sl = pl.ds(v * 16, 16)
                qsrc_v[sl] = qsrc_v[pl.ds(tb + v * 16, 16)]
                qdst_v[sl] = qdst_v[pl.ds(tb + v * 16, 16)]
            qn_s[0] = qn - tb
            return carry_c

        lax.fori_loop(0, NSCH, chunk_body, 0)
        qn = qn_s[0]

        ztr = jnp.full((16,), TRASH, jnp.int32)
        zsrc = jnp.zeros((16,), jnp.int32)
        for v in range(8):
            qsrc_v[pl.ds(qn + v * 16, 16)] = zsrc
            qdst_v[pl.ds(qn + v * 16, 16)] = ztr
        lax.fori_loop(0, (qn + GCH - 1) // GCH, flush_group, 0)

        pltpu.sync_copy(acc_v.at[pl.ds(0, RB)], out_hbm.at[pl.ds(t * RB, RB)])

        @pl.when(t == NW - 1)
        def _():
            pltpu.sync_copy(acc_v.at[pl.ds(RB, N - NW * RB)],
                            out_hbm.at[pl.ds(NW * RB, N - NW * RB)])

    return _mp


BLK = 1000


def _layer_body(h_ref, p_ref, b_ref, W1_ref, b1_ref, W2_ref, b2_ref, Wj_ref,
                hn_ref, P_ref):
    i = pl.program_id(0)
    t = h_ref[...] + p_ref[...]
    a = jnp.maximum(jnp.dot(t, W1_ref[...],
                            preferred_element_type=jnp.float32) + b1_ref[...], 0.0)
    hn = jnp.maximum(jnp.dot(a, W2_ref[...],
                             preferred_element_type=jnp.float32) + b2_ref[...], 0.0)
    hn_ref[...] = hn
    hj = jnp.dot(hn, Wj_ref[...], preferred_element_type=jnp.float32)
    mask = (b_ref[0] == lax.broadcasted_iota(jnp.int32, (G, BLK), 0)
            ).astype(jnp.float32)
    Pc = jnp.dot(mask, hj, preferred_element_type=jnp.float32,
                 precision=lax.Precision.HIGHEST)

    @pl.when(i == 0)
    def _():
        P_ref[...] = Pc

    @pl.when(i > 0)
    def _():
        P_ref[...] = P_ref[...] + Pc


_layer = pl.pallas_call(
    _layer_body,
    grid=(N // BLK,),
    in_specs=[
        pl.BlockSpec((BLK, H), lambda i: (i, 0)),
        pl.BlockSpec((BLK, H), lambda i: (i, 0)),
        pl.BlockSpec((1, 1, BLK), lambda i: (i, 0, 0)),
        pl.BlockSpec((H, H), lambda i: (0, 0)),
        pl.BlockSpec((1, H), lambda i: (0, 0)),
        pl.BlockSpec((H, H), lambda i: (0, 0)),
        pl.BlockSpec((1, H), lambda i: (0, 0)),
        pl.BlockSpec((H, H), lambda i: (0, 0)),
    ],
    out_specs=[
        pl.BlockSpec((BLK, H), lambda i: (i, 0)),
        pl.BlockSpec((G, H), lambda i: (0, 0)),
    ],
    out_shape=[
        jax.ShapeDtypeStruct((N, H), jnp.float32),
        jax.ShapeDtypeStruct((G, H), jnp.float32),
    ],
)


def _head_body(P1_ref, P2_ref, P3_ref, b_ref, bjk_ref, Wc1_ref, bc1_ref,
               g_ref, bt_ref, Wc2_ref, bc2_ref, out_ref):
    mask = (b_ref[...] == lax.broadcasted_iota(jnp.int32, (G, N), 0)
            ).astype(jnp.float32)
    counts = jnp.sum(mask, axis=1, keepdims=True)
    pooled = P1_ref[...] + P2_ref[...] + P3_ref[...] + counts * bjk_ref[...]
    z = jnp.dot(pooled, Wc1_ref[...], preferred_element_type=jnp.float32) + bc1_ref[...]
    mean = jnp.mean(z, axis=0, keepdims=True)
    var = jnp.mean((z - mean) ** 2, axis=0, keepdims=True)
    z = (z - mean) / jnp.sqrt(var + 1e-5) * g_ref[...] + bt_ref[...]
    z = jnp.maximum(z, 0.0)
    out_ref[...] = jnp.dot(z, Wc2_ref[...], preferred_element_type=jnp.float32) + bc2_ref[...]


_head = pl.pallas_call(
    _head_body,
    out_shape=jax.ShapeDtypeStruct((G, OUT), jnp.float32),
)


def kernel(x, edge_index, batch,
           W1_0, b1_0, W2_0, b2_0,
           W1_1, b1_1, W2_1, b2_1,
           W1_2, b1_2, W2_2, b2_2,
           Wjk, bjk, Wc1, bc1, bn_gamma, bn_beta, Wc2, bc2):
    src = edge_index[0]
    dst = edge_index[1]
    b2d = batch.reshape(1, N)
    b3d = batch.reshape(N // BLK, 1, BLK)
    mp = _get_mp()
    params = [(W1_0, b1_0, W2_0, b2_0),
              (W1_1, b1_1, W2_1, b2_1),
              (W1_2, b1_2, W2_2, b2_2)]
    h = x
    Ps = []
    for l, (W1, b1, W2, b2) in enumerate(params):
        agg = mp(h, src, dst)
        h, P = _layer(h, agg, b3d, W1, b1.reshape(1, H), W2, b2.reshape(1, H),
                      Wjk[l * H:(l + 1) * H])
        Ps.append(P)
    return _head(Ps[0], Ps[1], Ps[2], b2d, bjk.reshape(1, H),
                 Wc1, bc1.reshape(1, H), bn_gamma.reshape(1, H),
                 bn_beta.reshape(1, H), Wc2, bc2.reshape(1, OUT))


# speedup vs baseline: 1.2166x; 1.0576x over previous
import functools

import jax
import jax.numpy as jnp
from jax import lax
from jax.experimental import pallas as pl
from jax.experimental.pallas import tpu as pltpu
from jax.experimental.pallas import tpu_sc as plsc

N, E, D, H, OUT, G = 10000, 320000, 128, 128, 128, 64
NC, NS = 2, 16
NW = NC * NS
RB = 312
ACC_ROWS = 336
TRASH = 328
SCH = 2000
NSCH = E // SCH
GCH = 128
QCAP = SCH + GCH + 64
DUMP = QCAP - 16

@functools.lru_cache(maxsize=1)
def _get_mp():
    mesh = plsc.VectorSubcoreMesh(core_axis_name="c", subcore_axis_name="s",
                                  num_cores=NC, num_subcores=NS)

    @functools.partial(
        pl.kernel,
        out_type=jax.ShapeDtypeStruct((N, H), jnp.float32),
        mesh=mesh,
        scratch_types=[
            pltpu.VMEM((2 * SCH,), jnp.int32),
            pltpu.VMEM((2 * SCH,), jnp.int32),
            pltpu.VMEM((QCAP,), jnp.int32),
            pltpu.VMEM((GCH,), jnp.int32),
            pltpu.VMEM((GCH,), jnp.int32),
            pltpu.VMEM((GCH, H), jnp.float32),
            pltpu.VMEM((ACC_ROWS, H), jnp.float32),
            pltpu.SMEM((1,), jnp.int32),
            pltpu.SemaphoreType.DMA,
        ],
    )
    def _mp(h_hbm, src_hbm, dst_hbm, out_hbm, dstb_v, srcb_v, qp_v, gsrc_v,
            gdst_v, rows_v, acc_v, qn_s, sem):
        c = lax.axis_index("c")
        s = lax.axis_index("s")
        t = c * NS + s
        lo = t * RB
        hi = lo + RB + jnp.where(t == NW - 1, N - NW * RB, 0)
        z16 = jnp.zeros((16,), jnp.float32)

        def zero_body(i, carry):
            for v in range(8):
                acc_v[i, pl.ds(v * 16, 16)] = z16
            return carry

        lax.fori_loop(0, ACC_ROWS, zero_body, 0)
        qn_s[0] = 0

        def flush_group(g, carry):
            for u in range(GCH // 16):
                qp = qp_v[pl.ds(g * GCH + u * 16, 16)]
                gsrc_v[pl.ds(u * 16, 16)] = qp >> 9
                gdst_v[pl.ds(u * 16, 16)] = qp & 511
            pltpu.async_copy(h_hbm.at[gsrc_v], rows_v, sem).wait()

            def add_body(i, carry2):
                r = gdst_v[pl.ds(i, 16)][0]
                for v in range(8):
                    sl = pl.ds(v * 16, 16)
                    acc_v[r, sl] = acc_v[r, sl] + rows_v[i, sl]
                return carry2

            lax.fori_loop(0, GCH, add_body, 0)
            return carry

        def chunk_body(ci, carry_c):
            b = lax.rem(ci, 2)
            off = ci * SCH
            pltpu.sync_copy(dst_hbm.at[pl.ds(off, SCH)], dstb_v.at[pl.ds(b * SCH, SCH)])
            pltpu.sync_copy(src_hbm.at[pl.ds(off, SCH)], srcb_v.at[pl.ds(b * SCH, SCH)])

            spanu = (hi - lo).astype(jnp.uint32)

            def scan_body(k, carry):
                base = b * SCH + k * 16
                dv = dstb_v[pl.ds(base, 16)]
                sv = srcb_v[pl.ds(base, 16)]
                mi = jnp.where((dv - lo).astype(jnp.uint32) < spanu, 1, 0)
                qpv = jnp.left_shift(sv, 9) | (dv - lo)
                q = qn_s[0]
                for v in range(16):
                    miv = mi[v]
                    qidx = jnp.where(miv > 0, q, DUMP)
                    qp_v[pl.ds(qidx, 16)] = jnp.full((16,), qpv[v], jnp.int32)
                    q = q + miv
                qn_s[0] = q
                return carry

            lax.fori_loop(0, SCH // 16, scan_body, 0)
            qn = qn_s[0]
            groups = qn // GCH
            lax.fori_loop(0, groups, flush_group, 0)
            tb = groups * GCH
            for v in range(8):
                qp_v[pl.ds(v * 16, 16)] = qp_v[pl.ds(tb + v * 16, 16)]
            qn_s[0] = qn - tb
            return carry_c

        lax.fori_loop(0, NSCH, chunk_body, 0)
        qn = qn_s[0]

        ztr = jnp.full((16,), TRASH, jnp.int32)
        for v in range(8):
            qp_v[pl.ds(qn + v * 16, 16)] = ztr
        lax.fori_loop(0, (qn + GCH - 1) // GCH, flush_group, 0)

        pltpu.sync_copy(acc_v.at[pl.ds(0, RB)], out_hbm.at[pl.ds(t * RB, RB)])

        @pl.when(t == NW - 1)
        def _():
            pltpu.sync_copy(acc_v.at[pl.ds(RB, N - NW * RB)],
                            out_hbm.at[pl.ds(NW * RB, N - NW * RB)])

    return _mp


BLK = 1000


def _layer_body(h_ref, p_ref, b_ref, W1_ref, b1_ref, W2_ref, b2_ref, Wj_ref,
                hn_ref, P_ref):
    i = pl.program_id(0)
    t = h_ref[...] + p_ref[...]
    a = jnp.maximum(jnp.dot(t, W1_ref[...],
                            preferred_element_type=jnp.float32) + b1_ref[...], 0.0)
    hn = jnp.maximum(jnp.dot(a, W2_ref[...],
                             preferred_element_type=jnp.float32) + b2_ref[...], 0.0)
    hn_ref[...] = hn
    hj = jnp.dot(hn, Wj_ref[...], preferred_element_type=jnp.float32)
    mask = (b_ref[0] == lax.broadcasted_iota(jnp.int32, (G, BLK), 0)
            ).astype(jnp.float32)
    Pc = jnp.dot(mask, hj, preferred_element_type=jnp.float32,
                 precision=lax.Precision.HIGHEST)

    @pl.when(i == 0)
    def _():
        P_ref[...] = Pc

    @pl.when(i > 0)
    def _():
        P_ref[...] = P_ref[...] + Pc


_layer = pl.pallas_call(
    _layer_body,
    grid=(N // BLK,),
    in_specs=[
        pl.BlockSpec((BLK, H), lambda i: (i, 0)),
        pl.BlockSpec((BLK, H), lambda i: (i, 0)),
        pl.BlockSpec((1, 1, BLK), lambda i: (i, 0, 0)),
        pl.BlockSpec((H, H), lambda i: (0, 0)),
        pl.BlockSpec((1, H), lambda i: (0, 0)),
        pl.BlockSpec((H, H), lambda i: (0, 0)),
        pl.BlockSpec((1, H), lambda i: (0, 0)),
        pl.BlockSpec((H, H), lambda i: (0, 0)),
    ],
    out_specs=[
        pl.BlockSpec((BLK, H), lambda i: (i, 0)),
        pl.BlockSpec((G, H), lambda i: (0, 0)),
    ],
    out_shape=[
        jax.ShapeDtypeStruct((N, H), jnp.float32),
        jax.ShapeDtypeStruct((G, H), jnp.float32),
    ],
)


def _head_body(P1_ref, P2_ref, P3_ref, b_ref, bjk_ref, Wc1_ref, bc1_ref,
               g_ref, bt_ref, Wc2_ref, bc2_ref, out_ref):
    mask = (b_ref[...] == lax.broadcasted_iota(jnp.int32, (G, N), 0)
            ).astype(jnp.float32)
    counts = jnp.sum(mask, axis=1, keepdims=True)
    pooled = P1_ref[...] + P2_ref[...] + P3_ref[...] + counts * bjk_ref[...]
    z = jnp.dot(pooled, Wc1_ref[...], preferred_element_type=jnp.float32) + bc1_ref[...]
    mean = jnp.mean(z, axis=0, keepdims=True)
    var = jnp.mean((z - mean) ** 2, axis=0, keepdims=True)
    z = (z - mean) / jnp.sqrt(var + 1e-5) * g_ref[...] + bt_ref[...]
    z = jnp.maximum(z, 0.0)
    out_ref[...] = jnp.dot(z, Wc2_ref[...], preferred_element_type=jnp.float32) + bc2_ref[...]


_head = pl.pallas_call(
    _head_body,
    out_shape=jax.ShapeDtypeStruct((G, OUT), jnp.float32),
)


def kernel(x, edge_index, batch,
           W1_0, b1_0, W2_0, b2_0,
           W1_1, b1_1, W2_1, b2_1,
           W1_2, b1_2, W2_2, b2_2,
           Wjk, bjk, Wc1, bc1, bn_gamma, bn_beta, Wc2, bc2):
    src = edge_index[0]
    dst = edge_index[1]
    b2d = batch.reshape(1, N)
    b3d = batch.reshape(N // BLK, 1, BLK)
    mp = _get_mp()
    params = [(W1_0, b1_0, W2_0, b2_0),
              (W1_1, b1_1, W2_1, b2_1),
              (W1_2, b1_2, W2_2, b2_2)]
    h = x
    Ps = []
    for l, (W1, b1, W2, b2) in enumerate(params):
        agg = mp(h, src, dst)
        h, P = _layer(h, agg, b3d, W1, b1.reshape(1, H), W2, b2.reshape(1, H),
                      Wjk[l * H:(l + 1) * H])
        Ps.append(P)
    return _head(Ps[0], Ps[1], Ps[2], b2d, bjk.reshape(1, H),
                 Wc1, bc1.reshape(1, H), bn_gamma.reshape(1, H),
                 bn_beta.reshape(1, H), Wc2, bc2.reshape(1, OUT))
